# baseline (speedup 1.0000x reference)
import functools

import jax
import jax.numpy as jnp
from jax import lax
from jax.experimental import pallas as pl
from jax.experimental.pallas import tpu as pltpu
from jax.experimental.pallas import tpu_sc as plsc

MAX_OBJECTS = 16384
SLOTS = 8
D_MEMORY = 256
M = 4096

DM = SLOTS * D_MEMORY
DP = SLOTS * 4 * 4
DK = SLOTS // 4

NC = 2
NS = 16
NW = NC * NS
BPW = M // NW
CH = 16
NCH = BPW // CH


def _gather_body(idx_hbm, mem_hbm, pose_hbm, mask_hbm,
                 mem_out, pose_out, mask_out,
                 idx_v, mem_v, pose_v, mask_tab_v, mask_loc_v,
                 sem_g, sem_p, sem_k):
    wid = lax.axis_index("s") * NC + lax.axis_index("c")
    base = wid * BPW

    pltpu.sync_copy(idx_hbm.at[pl.ds(base, BPW)], idx_v)

    cp_p = pltpu.async_copy(pose_hbm.at[idx_v], pose_v, sem_p)
    cp_k = pltpu.async_copy(mask_hbm, mask_tab_v, sem_k)

    def start_gather(c, buf):
        return pltpu.async_copy(
            mem_hbm.at[idx_v.at[pl.ds(c * CH, CH)]], mem_v.at[buf], sem_g)

    cps = [start_gather(0, 0)]
    for c in range(NCH):
        cps[c].wait()
        if c + 1 < NCH:
            cps.append(start_gather(c + 1, (c + 1) % 2))
        pltpu.sync_copy(mem_v.at[c % 2], mem_out.at[pl.ds(base + c * CH, CH)])

    cp_p.wait()
    pltpu.sync_copy(pose_v, pose_out.at[pl.ds(base, BPW)])

    cp_k.wait()
    lane = lax.iota(jnp.int32, 16)
    for g in range(BPW // 16):
        obj = idx_v[pl.ds(g * 16, 16)]
        pos = g * 16 + lane
        for j in range(DK):
            vals = plsc.load_gather(mask_tab_v, [obj * DK + j])
            plsc.store_scatter(mask_loc_v, [pos * DK + j], vals)
    pltpu.sync_copy(mask_loc_v, mask_out.at[pl.ds(base * DK, BPW * DK)])


@jax.jit
def _retrieve(object_indices, memory2d, poses2d, mask2d):
    mesh = plsc.VectorSubcoreMesh(core_axis_name="c", subcore_axis_name="s")
    run = functools.partial(
        pl.kernel,
        out_type=(
            jax.ShapeDtypeStruct((M, DM), jnp.float32),
            jax.ShapeDtypeStruct((M, DP), jnp.float32),
            jax.ShapeDtypeStruct((M * DK,), jnp.int32),
        ),
        mesh=mesh,
        compiler_params=pltpu.CompilerParams(needs_layout_passes=False),
        scratch_types=[
            pltpu.VMEM((BPW,), jnp.int32),
            pltpu.VMEM((2, CH, DM), jnp.float32),
            pltpu.VMEM((BPW, DP), jnp.float32),
            pltpu.VMEM((MAX_OBJECTS * DK,), jnp.int32),
            pltpu.VMEM((BPW * DK,), jnp.int32),
            pltpu.SemaphoreType.DMA,
            pltpu.SemaphoreType.DMA,
            pltpu.SemaphoreType.DMA,
        ],
    )(_gather_body)
    return run(object_indices, memory2d, poses2d, mask2d)


def kernel(object_indices, memory, capture_poses, slot_filled):
    idx = object_indices.astype(jnp.int32)
    memory2d = memory.reshape(MAX_OBJECTS, DM)
    poses2d = capture_poses.reshape(MAX_OBJECTS, DP)
    mask2d = lax.bitcast_convert_type(
        slot_filled.astype(jnp.uint8).reshape(MAX_OBJECTS, DK, 4),
        jnp.int32).reshape(MAX_OBJECTS * DK)

    mem_o, pose_o, mask_o = _retrieve(idx, memory2d, poses2d, mask2d)

    memory_entries = mem_o.reshape(M, SLOTS, D_MEMORY)
    entry_poses = pose_o.reshape(M, SLOTS, 4, 4)
    entry_mask = lax.bitcast_convert_type(mask_o, jnp.uint8).reshape(M, SLOTS) != 0
    return (memory_entries, entry_poses, entry_mask)


# speedup vs baseline: 2.1334x; 2.1334x over previous
import functools

import jax
import jax.numpy as jnp
from jax import lax
from jax.experimental import pallas as pl
from jax.experimental.pallas import tpu as pltpu
from jax.experimental.pallas import tpu_sc as plsc

MAX_OBJECTS = 16384
SLOTS = 8
D_MEMORY = 256
M = 4096

DK = SLOTS // 4

NC = 2
NS = 16
NW = NC * NS
BPW = M // NW
CH = 16
NCH = BPW // CH


def _gather_body(idx_hbm, mem_hbm, pose_hbm, mask_hbm,
                 mem_out, pose_out, mask_out,
                 idx_v, mem_v, pose_v, mask_tab_v, mask_loc_v,
                 sem_g, sem_p, sem_k):
    wid = lax.axis_index("s") * NC + lax.axis_index("c")
    base = wid * BPW

    pltpu.sync_copy(idx_hbm.at[pl.ds(base, BPW)], idx_v)

    cp_p = pltpu.async_copy(pose_hbm.at[idx_v], pose_v, sem_p)
    cp_k = pltpu.async_copy(mask_hbm, mask_tab_v, sem_k)

    def start_gather(c, buf):
        return pltpu.async_copy(
            mem_hbm.at[idx_v.at[pl.ds(c * CH, CH)]], mem_v.at[buf], sem_g)

    cps = [start_gather(0, 0)]
    for c in range(NCH):
        cps[c].wait()
        if c + 1 < NCH:
            cps.append(start_gather(c + 1, (c + 1) % 2))
        pltpu.sync_copy(mem_v.at[c % 2], mem_out.at[pl.ds(base + c * CH, CH)])

    cp_p.wait()
    pltpu.sync_copy(pose_v, pose_out.at[pl.ds(base, BPW)])

    cp_k.wait()
    lane = lax.iota(jnp.int32, 16)
    for g in range(BPW // 16):
        obj = idx_v[pl.ds(g * 16, 16)]
        pos = g * 16 + lane
        for j in range(DK):
            vals = plsc.load_gather(mask_tab_v, [obj * DK + j])
            plsc.store_scatter(mask_loc_v, [pos * DK + j], vals)
    pltpu.sync_copy(mask_loc_v, mask_out.at[pl.ds(base * DK, BPW * DK)])


@jax.jit
def _retrieve(object_indices, memory, capture_poses, mask_words):
    mesh = plsc.VectorSubcoreMesh(core_axis_name="c", subcore_axis_name="s")
    run = functools.partial(
        pl.kernel,
        out_type=(
            jax.ShapeDtypeStruct((M, SLOTS, D_MEMORY), jnp.float32),
            jax.ShapeDtypeStruct((M, SLOTS * 16), jnp.float32),
            jax.ShapeDtypeStruct((M * DK,), jnp.int32),
        ),
        mesh=mesh,
        compiler_params=pltpu.CompilerParams(needs_layout_passes=False),
        scratch_types=[
            pltpu.VMEM((BPW,), jnp.int32),
            pltpu.VMEM((2, CH, SLOTS, D_MEMORY), jnp.float32),
            pltpu.VMEM((BPW, SLOTS * 16), jnp.float32),
            pltpu.VMEM((MAX_OBJECTS * DK,), jnp.int32),
            pltpu.VMEM((BPW * DK,), jnp.int32),
            pltpu.SemaphoreType.DMA,
            pltpu.SemaphoreType.DMA,
            pltpu.SemaphoreType.DMA,
        ],
    )(_gather_body)
    return run(object_indices, memory, capture_poses, mask_words)


def kernel(object_indices, memory, capture_poses, slot_filled):
    idx = object_indices.astype(jnp.int32)
    mask_words = lax.bitcast_convert_type(
        slot_filled.astype(jnp.uint8).reshape(MAX_OBJECTS, DK, 4),
        jnp.int32).reshape(MAX_OBJECTS * DK)

    poses2d = capture_poses.reshape(MAX_OBJECTS, SLOTS * 16)
    memory_entries, pose_o, mask_o = _retrieve(
        idx, memory, poses2d, mask_words)
    entry_poses = pose_o.reshape(M, SLOTS, 4, 4)

    entry_mask = lax.bitcast_convert_type(mask_o, jnp.uint8).reshape(M, SLOTS) != 0
    return (memory_entries, entry_poses, entry_mask)


# speedup vs baseline: 2.5666x; 1.2030x over previous
import functools

import jax
import jax.numpy as jnp
from jax import lax
from jax.experimental import pallas as pl
from jax.experimental.pallas import tpu as pltpu
from jax.experimental.pallas import tpu_sc as plsc

MAX_OBJECTS = 16384
SLOTS = 8
D_MEMORY = 256
M = 4096

NC = 2
NS = 16
NW = NC * NS
BPW = M // NW
CH = 8
NCH = BPW // CH
PHALF = MAX_OBJECTS // 2
NGRP = M // 16


def _gather_body(idx_hbm, mem_hbm, pose_hbm, mask_hbm,
                 mem_out, pose_out, mask_out,
                 idx_v, mem_v, pose_v, pout_v, mask_tab_v, mask_loc_v,
                 sem_g, sem_p, sem_k):
    wid = lax.axis_index("s") * NC + lax.axis_index("c")
    base = wid * BPW

    pltpu.sync_copy(idx_hbm, idx_v)

    cp_k = pltpu.async_copy(mask_hbm, mask_tab_v, sem_k)
    s_slot = wid // 4
    s_row = wid % 4
    cp_p = pltpu.async_copy(
        pose_hbm.at[s_slot, s_row, :, pl.ds(0, PHALF)], pose_v, sem_p)

    def start_gather(c, buf):
        return pltpu.async_copy(
            mem_hbm.at[idx_v.at[pl.ds(base + c * CH, CH)]],
            mem_v.at[buf], sem_g)

    cps = [start_gather(0, 0)]
    for c in range(NCH):
        cps[c].wait()
        if c + 1 < NCH:
            cps.append(start_gather(c + 1, (c + 1) % 2))
        pltpu.sync_copy(mem_v.at[c % 2], mem_out.at[pl.ds(base + c * CH, CH)])

    lane = lax.iota(jnp.int32, 16)
    for h in range(2):
        if h == 0:
            cp_p.wait()
        else:
            pltpu.async_copy(
                pose_hbm.at[s_slot, s_row, :, pl.ds(PHALF, PHALF)],
                pose_v, sem_p).wait()

        def pose_grp(g, _, h=h):
            obj = idx_v[pl.ds(g * 16, 16)]
            local = obj - h * PHALF
            ok = (local >= 0) & (local < PHALF)
            safe = jnp.clip(local, 0, PHALF - 1)
            pos = g * 16 + lane
            for b in range(4):
                bvec = jnp.full((16,), b, jnp.int32)
                vals = plsc.load_gather(pose_v, [bvec, safe])
                plsc.store_scatter(pout_v, [bvec, pos], vals, mask=ok)
            return 0

        lax.fori_loop(0, NGRP, pose_grp, 0)
    pltpu.sync_copy(pout_v, pose_out.at[s_slot, s_row])

    cp_k.wait()
    for g in range(BPW // 16):
        obj = idx_v[pl.ds(base + g * 16, 16)]
        vals = plsc.load_gather(mask_tab_v, [obj])
        plsc.store_scatter(mask_loc_v, [g * 16 + lane], vals)
    pltpu.sync_copy(mask_loc_v, mask_out.at[pl.ds(base, BPW)])


@jax.jit
def _retrieve(object_indices, memory, poses_t, mask_packed):
    mesh = plsc.VectorSubcoreMesh(core_axis_name="c", subcore_axis_name="s")
    run = functools.partial(
        pl.kernel,
        out_type=(
            jax.ShapeDtypeStruct((M, SLOTS, D_MEMORY), jnp.float32),
            jax.ShapeDtypeStruct((SLOTS, 4, 4, M), jnp.float32),
            jax.ShapeDtypeStruct((M,), jnp.int32),
        ),
        mesh=mesh,
        compiler_params=pltpu.CompilerParams(needs_layout_passes=False),
        scratch_types=[
            pltpu.VMEM((M,), jnp.int32),
            pltpu.VMEM((2, CH, SLOTS, D_MEMORY), jnp.float32),
            pltpu.VMEM((4, PHALF), jnp.float32),
            pltpu.VMEM((4, M), jnp.float32),
            pltpu.VMEM((MAX_OBJECTS,), jnp.int32),
            pltpu.VMEM((BPW,), jnp.int32),
            pltpu.SemaphoreType.DMA,
            pltpu.SemaphoreType.DMA,
            pltpu.SemaphoreType.DMA,
        ],
    )(_gather_body)
    return run(object_indices, memory, poses_t, mask_packed)


def kernel(object_indices, memory, capture_poses, slot_filled):
    idx = object_indices.astype(jnp.int32)
    poses_t = jnp.transpose(capture_poses, (1, 2, 3, 0))
    mask_packed = jnp.sum(
        slot_filled.astype(jnp.int32) << jnp.arange(SLOTS, dtype=jnp.int32),
        axis=1, dtype=jnp.int32)

    memory_entries, pose_o, mask_o = _retrieve(idx, memory, poses_t,
                                               mask_packed)

    entry_poses = jnp.transpose(pose_o, (3, 0, 1, 2))
    entry_mask = (
        (mask_o[:, None] >> jnp.arange(SLOTS, dtype=jnp.int32)) & 1) != 0
    return (memory_entries, entry_poses, entry_mask)


# speedup vs baseline: 2.9037x; 1.1314x over previous
import functools

import jax
import jax.numpy as jnp
from jax import lax
from jax.experimental import pallas as pl
from jax.experimental.pallas import tpu as pltpu
from jax.experimental.pallas import tpu_sc as plsc

MAX_OBJECTS = 16384
SLOTS = 8
D_MEMORY = 256
M = 4096

NC = 2
NS = 16
NW = NC * NS
BPW = M // NW
CH = 8
NCH = BPW // CH
KHALF = MAX_OBJECTS // 2
NGRP = M // 16
UNROLL = 4


def _gather_body(idx_hbm, mem_hbm, pose_hbm, mask_hbm,
                 mem_out, pose_out, mask_out,
                 idx_v, mem_v, pose_v, pout_v, mask_tab_v, mask_loc_v,
                 sem_g, sem_p, sem_k):
    wid = lax.axis_index("s") * NC + lax.axis_index("c")
    base = wid * BPW

    pltpu.sync_copy(idx_hbm, idx_v)

    s_slot = wid // 4
    s_row = wid % 4
    cp_p = pltpu.async_copy(pose_hbm.at[s_slot, s_row], pose_v, sem_p)
    cp_k = pltpu.async_copy(mask_hbm.at[pl.ds(0, KHALF)], mask_tab_v, sem_k)

    def start_gather(c, buf):
        return pltpu.async_copy(
            mem_hbm.at[idx_v.at[pl.ds(base + c * CH, CH)]],
            mem_v.at[buf], sem_g)

    cps = [start_gather(0, 0)]
    for c in range(NCH):
        cps[c].wait()
        if c + 1 < NCH:
            cps.append(start_gather(c + 1, (c + 1) % 2))
        pltpu.sync_copy(mem_v.at[c % 2], mem_out.at[pl.ds(base + c * CH, CH)])

    lane = lax.iota(jnp.int32, 16)
    cp_p.wait()

    def pose_grp(i, _):
        for u in range(UNROLL):
            g = i * UNROLL + u
            obj = idx_v[pl.ds(g * 16, 16)]
            pos = g * 16 + lane
            for b in range(4):
                bvec = jnp.full((16,), b, jnp.int32)
                vals = plsc.load_gather(pose_v, [bvec, obj])
                plsc.store_scatter(pout_v, [bvec, pos], vals)
        return 0

    lax.fori_loop(0, NGRP // UNROLL, pose_grp, 0)
    pltpu.sync_copy(pout_v, pose_out.at[s_slot, s_row])

    for h in range(2):
        if h == 0:
            cp_k.wait()
        else:
            pltpu.async_copy(mask_hbm.at[pl.ds(KHALF, KHALF)],
                             mask_tab_v, sem_k).wait()
        for g in range(BPW // 16):
            obj = idx_v[pl.ds(base + g * 16, 16)]
            local = obj - h * KHALF
            ok = (local >= 0) & (local < KHALF)
            safe = jnp.clip(local, 0, KHALF - 1)
            vals = plsc.load_gather(mask_tab_v, [safe])
            plsc.store_scatter(mask_loc_v, [g * 16 + lane], vals, mask=ok)
    pltpu.sync_copy(mask_loc_v, mask_out.at[pl.ds(base, BPW)])


@jax.jit
def _retrieve(object_indices, memory, poses_t, mask_packed):
    mesh = plsc.VectorSubcoreMesh(core_axis_name="c", subcore_axis_name="s")
    run = functools.partial(
        pl.kernel,
        out_type=(
            jax.ShapeDtypeStruct((M, SLOTS, D_MEMORY), jnp.float32),
            jax.ShapeDtypeStruct((SLOTS, 4, 4, M), jnp.float32),
            jax.ShapeDtypeStruct((M,), jnp.int32),
        ),
        mesh=mesh,
        compiler_params=pltpu.CompilerParams(needs_layout_passes=False),
        scratch_types=[
            pltpu.VMEM((M,), jnp.int32),
            pltpu.VMEM((2, CH, SLOTS, D_MEMORY), jnp.float32),
            pltpu.VMEM((4, MAX_OBJECTS), jnp.float32),
            pltpu.VMEM((4, M), jnp.float32),
            pltpu.VMEM((KHALF,), jnp.int32),
            pltpu.VMEM((BPW,), jnp.int32),
            pltpu.SemaphoreType.DMA,
            pltpu.SemaphoreType.DMA,
            pltpu.SemaphoreType.DMA,
        ],
    )(_gather_body)
    return run(object_indices, memory, poses_t, mask_packed)


def kernel(object_indices, memory, capture_poses, slot_filled):
    idx = object_indices.astype(jnp.int32)
    poses_t = jnp.transpose(capture_poses, (1, 2, 3, 0))
    mask_packed = jnp.sum(
        slot_filled.astype(jnp.int32) << jnp.arange(SLOTS, dtype=jnp.int32),
        axis=1, dtype=jnp.int32)

    memory_entries, pose_o, mask_o = _retrieve(idx, memory, poses_t,
                                               mask_packed)

    entry_poses = jnp.transpose(pose_o, (3, 0, 1, 2))
    entry_mask = (
        (mask_o[:, None] >> jnp.arange(SLOTS, dtype=jnp.int32)) & 1) != 0
    return (memory_entries, entry_poses, entry_mask)


# speedup vs baseline: 3.0610x; 1.0542x over previous
import functools

import jax
import jax.numpy as jnp
from jax import lax
from jax.experimental import pallas as pl
from jax.experimental.pallas import tpu as pltpu
from jax.experimental.pallas import tpu_sc as plsc

MAX_OBJECTS = 16384
SLOTS = 8
D_MEMORY = 256
M = 4096

NC = 2
NS = 16
NW = NC * NS
BPW = M // NW
CH = 8
NCH = BPW // CH
KHALF = MAX_OBJECTS // 2
NGRP = M // 16
UNROLL = 4


def _gather_body(idx_hbm, mem_hbm, pose_hbm, mask_hbm,
                 mem_out, pose_out, mask_out,
                 idx_v, mem_v, pose_v, pout_v, mask_tab_v, mask_loc_v,
                 sem_g, sem_o, sem_p, sem_k):
    wid = lax.axis_index("s") * NC + lax.axis_index("c")
    base = wid * BPW

    pltpu.sync_copy(idx_hbm, idx_v)

    s_slot = wid // 4
    s_row = wid % 4
    cp_p = pltpu.async_copy(pose_hbm.at[s_slot, s_row], pose_v, sem_p)
    cp_k = pltpu.async_copy(mask_hbm.at[pl.ds(0, KHALF)], mask_tab_v, sem_k)

    lane = lax.iota(jnp.int32, 16)

    def pose_range(lo, n):
        def body(i, _):
            for u in range(UNROLL):
                g = lo + i * UNROLL + u
                obj = idx_v[pl.ds(g * 16, 16)]
                pos = g * 16 + lane
                for b in range(4):
                    bvec = jnp.full((16,), b, jnp.int32)
                    vals = plsc.load_gather(pose_v, [bvec, obj])
                    plsc.store_scatter(pout_v, [bvec, pos], vals)
            return 0
        lax.fori_loop(0, n // UNROLL, body, 0)

    quota = [0, 0] + [20] * 8 + [16] * 6
    q_lo = [sum(quota[:c]) for c in range(NCH)]

    def start_gather(c, buf):
        return pltpu.async_copy(
            mem_hbm.at[idx_v.at[pl.ds(base + c * CH, CH)]],
            mem_v.at[buf], sem_g)

    cps = [start_gather(0, 0)]
    outs = [None] * NCH
    for c in range(NCH):
        cps[c].wait()
        if c + 1 < NCH:
            if c >= 1:
                outs[c - 1].wait()
            cps.append(start_gather(c + 1, (c + 1) % 2))
        outs[c] = pltpu.async_copy(
            mem_v.at[c % 2], mem_out.at[pl.ds(base + c * CH, CH)], sem_o)
        if quota[c]:
            if q_lo[c] == 0:
                cp_p.wait()
            pose_range(q_lo[c], quota[c])
    outs[NCH - 2].wait()
    outs[NCH - 1].wait()
    pltpu.sync_copy(pout_v, pose_out.at[s_slot, s_row])

    for h in range(2):
        if h == 0:
            cp_k.wait()
        else:
            pltpu.async_copy(mask_hbm.at[pl.ds(KHALF, KHALF)],
                             mask_tab_v, sem_k).wait()
        for g in range(BPW // 16):
            obj = idx_v[pl.ds(base + g * 16, 16)]
            local = obj - h * KHALF
            ok = (local >= 0) & (local < KHALF)
            safe = jnp.clip(local, 0, KHALF - 1)
            vals = plsc.load_gather(mask_tab_v, [safe])
            plsc.store_scatter(mask_loc_v, [g * 16 + lane], vals, mask=ok)
    pltpu.sync_copy(mask_loc_v, mask_out.at[pl.ds(base, BPW)])


@jax.jit
def _retrieve(object_indices, memory, poses_t, mask_packed):
    mesh = plsc.VectorSubcoreMesh(core_axis_name="c", subcore_axis_name="s")
    run = functools.partial(
        pl.kernel,
        out_type=(
            jax.ShapeDtypeStruct((M, SLOTS, D_MEMORY), jnp.float32),
            jax.ShapeDtypeStruct((SLOTS, 4, 4, M), jnp.float32),
            jax.ShapeDtypeStruct((M,), jnp.int32),
        ),
        mesh=mesh,
        compiler_params=pltpu.CompilerParams(needs_layout_passes=False),
        scratch_types=[
            pltpu.VMEM((M,), jnp.int32),
            pltpu.VMEM((2, CH, SLOTS, D_MEMORY), jnp.float32),
            pltpu.VMEM((4, MAX_OBJECTS), jnp.float32),
            pltpu.VMEM((4, M), jnp.float32),
            pltpu.VMEM((KHALF,), jnp.int32),
            pltpu.VMEM((BPW,), jnp.int32),
            pltpu.SemaphoreType.DMA,
            pltpu.SemaphoreType.DMA,
            pltpu.SemaphoreType.DMA,
            pltpu.SemaphoreType.DMA,
        ],
    )(_gather_body)
    return run(object_indices, memory, poses_t, mask_packed)


def kernel(object_indices, memory, capture_poses, slot_filled):
    idx = object_indices.astype(jnp.int32)
    poses_t = jnp.transpose(capture_poses, (1, 2, 3, 0))
    mask_packed = jnp.sum(
        slot_filled.astype(jnp.int32) << jnp.arange(SLOTS, dtype=jnp.int32),
        axis=1, dtype=jnp.int32)

    memory_entries, pose_o, mask_o = _retrieve(idx, memory, poses_t,
                                               mask_packed)

    entry_poses = jnp.transpose(pose_o, (3, 0, 1, 2))
    entry_mask = (
        (mask_o[:, None] >> jnp.arange(SLOTS, dtype=jnp.int32)) & 1) != 0
    return (memory_entries, entry_poses, entry_mask)
